# baseline (speedup 1.0000x reference)
import jax
import jax.numpy as jnp
from jax import lax
from jax.experimental import pallas as pl
from jax.experimental.pallas import tpu as pltpu

NUM_EMBEDDINGS = 1024
EMBEDDING_DIM = 64
COMMITMENT_COST = 0.1

TILE = 2048


def _vq_kernel(z_hbm, cb_ref, out_hbm, loss_ref,
               zbuf, obuf, acc_ref, in_sems, out_sems):
    i = pl.program_id(0)
    nsteps = pl.num_programs(0)
    slot = lax.rem(i, 2)
    nxt = lax.rem(i + 1, 2)

    @pl.when(i == 0)
    def _():
        acc_ref[0] = 0.0
        pltpu.make_async_copy(z_hbm.at[pl.ds(0, TILE)], zbuf.at[0],
                              in_sems.at[0]).start()

    @pl.when(i + 1 < nsteps)
    def _():
        pltpu.make_async_copy(z_hbm.at[pl.ds((i + 1) * TILE, TILE)],
                              zbuf.at[nxt], in_sems.at[nxt]).start()

    pltpu.make_async_copy(z_hbm.at[pl.ds(i * TILE, TILE)], zbuf.at[slot],
                          in_sems.at[slot]).wait()

    z = zbuf[slot]
    cb = cb_ref[...]

    z_sq = jnp.sum(z * z, axis=1, keepdims=True)
    cb_sq = jnp.sum(cb * cb, axis=1)
    cross = lax.dot_general(
        z, cb, dimension_numbers=(((1,), (1,)), ((), ())),
        preferred_element_type=jnp.float32)
    dist = (z_sq + cb_sq[None, :]) - 2.0 * cross

    min_d = jnp.min(dist, axis=1, keepdims=True)
    col_i = lax.broadcasted_iota(jnp.int32, dist.shape, 1)
    col_f = lax.bitcast_convert_type(col_i | jnp.int32(0x3F800000),
                                     jnp.float32)
    idx_f = jnp.min(jnp.where(dist == min_d, col_f, jnp.float32(2.0)),
                    axis=1, keepdims=True)

    onehot = (col_f == idx_f).astype(jnp.float32)
    zq = lax.dot_general(
        onehot, cb, dimension_numbers=(((1,), (0,)), ((), ())),
        preferred_element_type=jnp.float32)

    @pl.when(i >= 2)
    def _():
        pltpu.make_async_copy(obuf.at[slot],
                              out_hbm.at[pl.ds((i - 2) * TILE, TILE)],
                              out_sems.at[slot]).wait()

    obuf[slot] = zq
    pltpu.make_async_copy(obuf.at[slot], out_hbm.at[pl.ds(i * TILE, TILE)],
                          out_sems.at[slot]).start()

    acc_ref[0] += jnp.sum(min_d)

    @pl.when(i == nsteps - 1)
    def _():
        mean_sq = acc_ref[0] / (nsteps * TILE * EMBEDDING_DIM)
        loss_ref[0, 0] = mean_sq + COMMITMENT_COST * mean_sq
        pltpu.make_async_copy(obuf.at[slot],
                              out_hbm.at[pl.ds(i * TILE, TILE)],
                              out_sems.at[slot]).wait()
        pltpu.make_async_copy(obuf.at[nxt],
                              out_hbm.at[pl.ds((i - 1) * TILE, TILE)],
                              out_sems.at[nxt]).wait()


@jax.jit
def kernel(z, codebook):
    zz = z[0]
    n = zz.shape[0] * zz.shape[1]
    z_flat = zz.reshape(n, EMBEDDING_DIM)
    grid = n // TILE

    out, loss = pl.pallas_call(
        _vq_kernel,
        grid=(grid,),
        in_specs=[
            pl.BlockSpec(memory_space=pl.ANY),
            pl.BlockSpec((NUM_EMBEDDINGS, EMBEDDING_DIM), lambda i: (0, 0)),
        ],
        out_specs=[
            pl.BlockSpec(memory_space=pl.ANY),
            pl.BlockSpec((1, 1), lambda i: (0, 0), memory_space=pltpu.SMEM),
        ],
        out_shape=[
            jax.ShapeDtypeStruct((n, EMBEDDING_DIM), jnp.float32),
            jax.ShapeDtypeStruct((1, 1), jnp.float32),
        ],
        scratch_shapes=[
            pltpu.VMEM((2, TILE, EMBEDDING_DIM), jnp.float32),
            pltpu.VMEM((2, TILE, EMBEDDING_DIM), jnp.float32),
            pltpu.SMEM((1,), jnp.float32),
            pltpu.SemaphoreType.DMA((2,)),
            pltpu.SemaphoreType.DMA((2,)),
        ],
    )(z_flat, codebook)

    return (out.reshape(zz.shape), loss[0, 0])


# speedup vs baseline: 1.0913x; 1.0913x over previous
import jax
import jax.numpy as jnp
from jax import lax
from jax.experimental import pallas as pl
from jax.experimental.pallas import tpu as pltpu

NUM_EMBEDDINGS = 1024
EMBEDDING_DIM = 64
COMMITMENT_COST = 0.1

TILE = 4608


def _vq_kernel(z_ref, cb_ref, out_ref, loss_ref, acc_ref):
    i = pl.program_id(0)
    nsteps = pl.num_programs(0)
    z = z_ref[...]
    cb = cb_ref[...]

    z_sq = jnp.sum(z * z, axis=1, keepdims=True)
    cb_sq = jnp.sum(cb * cb, axis=1)
    cross = lax.dot_general(
        z, cb, dimension_numbers=(((1,), (1,)), ((), ())),
        preferred_element_type=jnp.float32)
    dist = (z_sq + cb_sq[None, :]) - 2.0 * cross

    min_d = jnp.min(dist, axis=1, keepdims=True)
    col_i = lax.broadcasted_iota(jnp.int32, dist.shape, 1)
    col_f = lax.bitcast_convert_type(col_i | jnp.int32(0x3F800000),
                                     jnp.float32)
    idx_f = jnp.min(jnp.where(dist == min_d, col_f, jnp.float32(2.0)),
                    axis=1, keepdims=True)

    onehot = (col_f == idx_f).astype(jnp.float32)
    out_ref[...] = lax.dot_general(
        onehot, cb, dimension_numbers=(((1,), (0,)), ((), ())),
        preferred_element_type=jnp.float32)

    @pl.when(i == 0)
    def _():
        acc_ref[0] = 0.0

    acc_ref[0] += jnp.sum(min_d)

    @pl.when(i == nsteps - 1)
    def _():
        mean_sq = acc_ref[0] / (nsteps * TILE * EMBEDDING_DIM)
        loss_ref[0, 0] = mean_sq + COMMITMENT_COST * mean_sq


@jax.jit
def kernel(z, codebook):
    zz = z[0]
    n = zz.shape[0] * zz.shape[1]
    z_flat = zz.reshape(n, EMBEDDING_DIM)
    grid = n // TILE

    out, loss = pl.pallas_call(
        _vq_kernel,
        grid=(grid,),
        in_specs=[
            pl.BlockSpec((TILE, EMBEDDING_DIM), lambda i: (i, 0)),
            pl.BlockSpec((NUM_EMBEDDINGS, EMBEDDING_DIM), lambda i: (0, 0)),
        ],
        out_specs=[
            pl.BlockSpec((TILE, EMBEDDING_DIM), lambda i: (i, 0)),
            pl.BlockSpec((1, 1), lambda i: (0, 0), memory_space=pltpu.SMEM),
        ],
        out_shape=[
            jax.ShapeDtypeStruct((n, EMBEDDING_DIM), jnp.float32),
            jax.ShapeDtypeStruct((1, 1), jnp.float32),
        ],
        scratch_shapes=[pltpu.SMEM((1,), jnp.float32)],
    )(z_flat, codebook)

    return (out.reshape(zz.shape), loss[0, 0])


# speedup vs baseline: 1.0927x; 1.0012x over previous
import jax
import jax.numpy as jnp
from jax import lax
from jax.experimental import pallas as pl
from jax.experimental.pallas import tpu as pltpu

NUM_EMBEDDINGS = 1024
EMBEDDING_DIM = 64
COMMITMENT_COST = 0.1

TILE = 6144


def _vq_kernel(z_ref, cb_ref, out_ref, loss_ref, acc_ref):
    i = pl.program_id(0)
    nsteps = pl.num_programs(0)
    z = z_ref[...]
    cb = cb_ref[...]

    z_sq = jnp.sum(z * z, axis=1, keepdims=True)
    cb_sq = jnp.sum(cb * cb, axis=1)
    cross = lax.dot_general(
        z, cb, dimension_numbers=(((1,), (1,)), ((), ())),
        preferred_element_type=jnp.float32)
    dist = (z_sq + cb_sq[None, :]) - 2.0 * cross

    min_d = jnp.min(dist, axis=1, keepdims=True)
    col_i = lax.broadcasted_iota(jnp.int32, dist.shape, 1)
    col_f = lax.bitcast_convert_type(col_i | jnp.int32(0x3F800000),
                                     jnp.float32)
    idx_f = jnp.min(jnp.where(dist == min_d, col_f, jnp.float32(2.0)),
                    axis=1, keepdims=True)

    onehot = (col_f == idx_f).astype(jnp.float32)
    out_ref[...] = lax.dot_general(
        onehot, cb, dimension_numbers=(((1,), (0,)), ((), ())),
        preferred_element_type=jnp.float32)

    @pl.when(i == 0)
    def _():
        acc_ref[0] = 0.0

    acc_ref[0] += jnp.sum(min_d)

    @pl.when(i == nsteps - 1)
    def _():
        mean_sq = acc_ref[0] / (nsteps * TILE * EMBEDDING_DIM)
        loss_ref[0, 0] = mean_sq + COMMITMENT_COST * mean_sq


@jax.jit
def kernel(z, codebook):
    zz = z[0]
    n = zz.shape[0] * zz.shape[1]
    z_flat = zz.reshape(n, EMBEDDING_DIM)
    grid = n // TILE

    out, loss = pl.pallas_call(
        _vq_kernel,
        grid=(grid,),
        in_specs=[
            pl.BlockSpec((TILE, EMBEDDING_DIM), lambda i: (i, 0)),
            pl.BlockSpec((NUM_EMBEDDINGS, EMBEDDING_DIM), lambda i: (0, 0)),
        ],
        out_specs=[
            pl.BlockSpec((TILE, EMBEDDING_DIM), lambda i: (i, 0)),
            pl.BlockSpec((1, 1), lambda i: (0, 0), memory_space=pltpu.SMEM),
        ],
        out_shape=[
            jax.ShapeDtypeStruct((n, EMBEDDING_DIM), jnp.float32),
            jax.ShapeDtypeStruct((1, 1), jnp.float32),
        ],
        scratch_shapes=[pltpu.SMEM((1,), jnp.float32)],
    )(z_flat, codebook)

    return (out.reshape(zz.shape), loss[0, 0])
